# baseline (speedup 1.0000x reference)
import functools

import jax
import jax.numpy as jnp
from jax import lax
from jax.experimental import pallas as pl
from jax.experimental.pallas import tpu as pltpu
from jax.experimental.pallas import tpu_sc as plsc

NC = 2
NS = 16
L = 16
NW = NC * NS
D = 768
K = 4
C = 8
RING = 8
LOOK = 6
TCR = 512


def _sc_gather(idx_k, table, rows_k):
    per_w = rows_k // NW
    nchunk = per_w // C
    nround = nchunk // RING

    mesh = plsc.VectorSubcoreMesh(core_axis_name="c", subcore_axis_name="s")

    @functools.partial(
        pl.kernel,
        out_type=jax.ShapeDtypeStruct((rows_k, D), jnp.float32),
        mesh=mesh,
        scratch_types=[
            pltpu.VMEM((per_w,), jnp.int32),
            pltpu.VMEM((RING, C, D), jnp.float32),
            pltpu.SemaphoreType.DMA((RING,)),
            pltpu.SemaphoreType.DMA((RING,)),
        ],
    )
    def gather_kernel(idx_hbm, tab_hbm, emb_hbm, idx_v, rows_v, sem_g, sem_o):
        wid = lax.axis_index("s") * NC + lax.axis_index("c")
        base = wid * per_w
        pltpu.sync_copy(idx_hbm.at[pl.ds(base, per_w)], idx_v)

        def issue_in(chunk, slot):
            pltpu.async_copy(
                tab_hbm.at[idx_v.at[pl.ds(chunk * C, C)]], rows_v.at[slot],
                sem_g.at[slot])

        def wait_in(chunk, slot):
            pltpu.make_async_copy(
                tab_hbm.at[idx_v.at[pl.ds(chunk * C, C)]], rows_v.at[slot],
                sem_g.at[slot]).wait()

        def issue_out(chunk, slot):
            pltpu.async_copy(
                rows_v.at[slot], emb_hbm.at[pl.ds(base + chunk * C, C)],
                sem_o.at[slot])

        def wait_out(chunk, slot):
            pltpu.make_async_copy(
                rows_v.at[slot], emb_hbm.at[pl.ds(base + chunk * C, C)],
                sem_o.at[slot]).wait()

        for b in range(LOOK):
            issue_in(b, b)

        @pl.loop(0, nround)
        def _(r):
            for b in range(RING):
                i = r * RING + b
                wait_in(i, b)
                issue_out(i, b)
                j_slot = (b + LOOK) % RING
                if b < RING - LOOK:
                    @pl.when(r > 0)
                    def _():
                        wait_out((r - 1) * RING + b + LOOK, j_slot)
                    issue_in(i + LOOK, j_slot)
                else:
                    @pl.when(r < nround - 1)
                    def _():
                        wait_out(r * RING + b + LOOK - RING, j_slot)
                        issue_in(i + LOOK, j_slot)

        for b in range(RING):
            wait_out((nround - 1) * RING + b, b)

    return gather_kernel(idx_k, table)


def _tc_add_chunk(out_prev, x2, emb_k, k, nblk):
    def add_body(_, x_ref, emb_ref, out_ref):
        out_ref[...] = x_ref[...] + emb_ref[...]

    return pl.pallas_call(
        add_body,
        grid=(nblk,),
        in_specs=[
            pl.BlockSpec(memory_space=pl.ANY),
            pl.BlockSpec((TCR, D), lambda i, k=k: (k * nblk + i, 0)),
            pl.BlockSpec((TCR, D), lambda i: (i, 0)),
        ],
        out_specs=pl.BlockSpec((TCR, D), lambda i, k=k: (k * nblk + i, 0)),
        out_shape=jax.ShapeDtypeStruct(x2.shape, jnp.float32),
        input_output_aliases={0: 0},
    )(out_prev, x2, emb_k)


def kernel(x, positions, table):
    bt, s, d = x.shape
    B = bt * s
    x2 = x.reshape(B, d)
    idx = positions.reshape(B).astype(jnp.int32)

    rows_k = B // K
    nblk = rows_k // TCR

    embs = [_sc_gather(idx[k * rows_k:(k + 1) * rows_k], table, rows_k)
            for k in range(K)]

    def first_body(x_ref, emb_ref, out_ref):
        out_ref[...] = x_ref[...] + emb_ref[...]

    out = pl.pallas_call(
        first_body,
        grid=(nblk,),
        in_specs=[
            pl.BlockSpec((TCR, D), lambda i: (i, 0)),
            pl.BlockSpec((TCR, D), lambda i: (i, 0)),
        ],
        out_specs=pl.BlockSpec((TCR, D), lambda i: (i, 0)),
        out_shape=jax.ShapeDtypeStruct((B, d), jnp.float32),
    )(x2, embs[0])
    for k in range(1, K):
        out = _tc_add_chunk(out, x2, embs[k], k, nblk)

    return out.reshape(bt, s, d)


# speedup vs baseline: 1.5284x; 1.5284x over previous
import functools

import jax
import jax.numpy as jnp
from jax import lax
from jax.experimental import pallas as pl
from jax.experimental.pallas import tpu as pltpu
from jax.experimental.pallas import tpu_sc as plsc

NC = 2
NS = 16
L = 16
NW = NC * NS
D = 768
VPR = D // L
C = 8
RING = 8
LOOK = 6


def _emb_add(x2, idx, table):
    B = x2.shape[0]
    per_w = B // NW
    nchunk = per_w // C
    nround = nchunk // RING

    mesh = plsc.VectorSubcoreMesh(core_axis_name="c", subcore_axis_name="s")

    @functools.partial(
        pl.kernel,
        out_type=jax.ShapeDtypeStruct((B, D), jnp.float32),
        mesh=mesh,
        scratch_types=[
            pltpu.VMEM((per_w,), jnp.int32),
            pltpu.VMEM((RING, C, D), jnp.float32),
            pltpu.VMEM((RING, C, D), jnp.float32),
            pltpu.SemaphoreType.DMA((RING,)),
            pltpu.SemaphoreType.DMA((RING,)),
            pltpu.SemaphoreType.DMA((RING,)),
        ],
    )
    def emb_add_kernel(x_hbm, idx_hbm, tab_hbm, out_hbm,
                       idx_v, x_v, rows_v, sem_x, sem_g, sem_o):
        wid = lax.axis_index("s") * NC + lax.axis_index("c")
        base = wid * per_w
        pltpu.sync_copy(idx_hbm.at[pl.ds(base, per_w)], idx_v)

        def issue_in(chunk, slot):
            pltpu.async_copy(
                x_hbm.at[pl.ds(base + chunk * C, C)], x_v.at[slot],
                sem_x.at[slot])
            pltpu.async_copy(
                tab_hbm.at[idx_v.at[pl.ds(chunk * C, C)]], rows_v.at[slot],
                sem_g.at[slot])

        def wait_in(chunk, slot):
            pltpu.make_async_copy(
                x_hbm.at[pl.ds(base + chunk * C, C)], x_v.at[slot],
                sem_x.at[slot]).wait()
            pltpu.make_async_copy(
                tab_hbm.at[idx_v.at[pl.ds(chunk * C, C)]], rows_v.at[slot],
                sem_g.at[slot]).wait()

        def issue_out(chunk, slot):
            pltpu.async_copy(
                rows_v.at[slot], out_hbm.at[pl.ds(base + chunk * C, C)],
                sem_o.at[slot])

        def wait_out(chunk, slot):
            pltpu.make_async_copy(
                rows_v.at[slot], out_hbm.at[pl.ds(base + chunk * C, C)],
                sem_o.at[slot]).wait()

        for b in range(LOOK):
            issue_in(b, b)

        @pl.loop(0, nround)
        def _(r):
            for b in range(RING):
                i = r * RING + b
                wait_in(i, b)

                @pl.loop(0, C)
                def _(row):
                    @pl.loop(0, VPR, unroll=16)
                    def _(v):
                        sl = pl.ds(v * L, L)
                        plsc.addupdate(rows_v.at[b, row, sl], x_v[b, row, sl])

                issue_out(i, b)

                j_slot = (b + LOOK) % RING
                if b < RING - LOOK:
                    @pl.when(r > 0)
                    def _():
                        wait_out((r - 1) * RING + b + LOOK, j_slot)
                    issue_in(i + LOOK, j_slot)
                else:
                    @pl.when(r < nround - 1)
                    def _():
                        wait_out(r * RING + b + LOOK - RING, j_slot)
                        issue_in(i + LOOK, j_slot)

        for b in range(RING):
            wait_out((nround - 1) * RING + b, b)

    return emb_add_kernel(x2, idx, table)


def kernel(x, positions, table):
    bt, s, d = x.shape
    x2 = x.reshape(bt * s, d)
    idx = positions.reshape(bt * s).astype(jnp.int32)
    out = _emb_add(x2, idx, table)
    return out.reshape(bt, s, d)
